# baseline (speedup 1.0000x reference)
import functools

import jax
import jax.numpy as jnp
from jax import lax
from jax.experimental import pallas as pl
from jax.experimental.pallas import tpu as pltpu
from jax.experimental.pallas import tpu_sc as plsc

N = 10000
E = 320000
F = 128
H = 16
L = 64

NPAD = 10240
NW = 32
EPW = E // NW
CHUNK = 80
NCH = EPW // CHUNK
RPT = NPAD // 16
NBUF = 5
NRING = NCH // NBUF
ZR = 40
TAIL = N - 15 * RPT

_MESH = dict(core_axis_name="c", subcore_axis_name="s")


@functools.partial(
    pl.kernel,
    out_type=[
        jax.ShapeDtypeStruct((2, N), jnp.float32),
        jax.ShapeDtypeStruct((NW, NCH, CHUNK), jnp.int32),
    ],
    mesh=plsc.VectorSubcoreMesh(**_MESH),
    compiler_params=pltpu.CompilerParams(use_tc_tiling_on_sc=False),
    scratch_types=[
        pltpu.VMEM((EPW,), jnp.int32),
        pltpu.VMEM((EPW,), jnp.int32),
        pltpu.VMEM((NCH, CHUNK), jnp.int32),
        pltpu.VMEM((CHUNK,), jnp.float32),
        pltpu.VMEM((RPT,), jnp.float32),
        pltpu.VMEM_SHARED((NPAD,), jnp.float32),
        pltpu.SemaphoreType.DMA,
    ],
)
def _deg_adj(ei_hbm, deg_hbm, adj_hbm,
             frm_v, to_v, adj_v, ones_v, zero_v, acc_sh, dsem):
    cid = lax.axis_index("c")
    sid = lax.axis_index("s")
    wid = cid * 16 + sid
    z16 = jnp.zeros((16,), jnp.float32)
    o16 = jnp.ones((16,), jnp.float32)

    def fill_z(i, _):
        zero_v[pl.ds(i * 16, 16)] = z16
        return 0
    lax.fori_loop(0, RPT // 16, fill_z, 0)

    def fill_o(i, _):
        ones_v[pl.ds(i * 16, 16)] = o16
        return 0
    lax.fori_loop(0, CHUNK // 16, fill_o, 0)

    pltpu.sync_copy(zero_v, acc_sh.at[pl.ds(sid * RPT, RPT)])
    pltpu.sync_copy(ei_hbm.at[0, pl.ds(wid * EPW, EPW)], frm_v)
    pltpu.sync_copy(ei_hbm.at[1, pl.ds(wid * EPW, EPW)], to_v)
    plsc.subcore_barrier()

    def chunk(j, _):
        dump = N + ((wid * NCH + j) * 7) % (NPAD - N)

        def vec(k, _):
            f16 = frm_v[pl.ds(j * CHUNK + k * 16, 16)]
            t16 = to_v[pl.ds(j * CHUNK + k * 16, 16)]
            adj_v[j, pl.ds(k * 16, 16)] = jnp.where(f16 != t16, t16, dump)
            return 0
        lax.fori_loop(0, CHUNK // 16, vec, 0)
        pltpu.async_copy(ones_v, acc_sh.at[adj_v.at[j]], dsem, add=True)
        return 0
    lax.fori_loop(0, NCH, chunk, 0)

    pltpu.sync_copy(adj_v, adj_hbm.at[wid])

    def drain(j, _):
        pltpu.make_async_copy(ones_v, acc_sh.at[adj_v.at[j]], dsem).wait()
        return 0
    lax.fori_loop(0, NCH, drain, 0)
    plsc.subcore_barrier()

    @pl.when(sid < 15)
    def _():
        pltpu.sync_copy(acc_sh.at[pl.ds(sid * RPT, RPT)],
                        deg_hbm.at[cid, pl.ds(sid * RPT, RPT)])

    @pl.when(sid == 15)
    def _():
        pltpu.sync_copy(acc_sh.at[pl.ds(15 * RPT, TAIL)],
                        deg_hbm.at[cid, pl.ds(15 * RPT, TAIL)])


EPT = 2 * EPW
NCH2 = 2 * NCH
NRING2 = NCH2 // NBUF


def _ring(src, frm_v, adj_v, bufs, gsems, ssems, acc_sh, nch, off):
    def idx(j):
        return frm_v.at[pl.ds((off + j) * CHUNK, CHUNK)]

    def arow(j):
        return acc_sh.at[adj_v.at[off + j]]

    for b in range(NBUF):
        pltpu.async_copy(src.at[idx(b)], bufs.at[b], gsems[b])

    def ring(g, _):
        j0 = g * NBUF
        for b in range(NBUF):
            pltpu.make_async_copy(src.at[idx(j0 + b)], bufs.at[b],
                                  gsems[b]).wait()
            pltpu.async_copy(bufs.at[b], arow(j0 + b), ssems[b], add=True)
            if b > 0:
                pltpu.make_async_copy(bufs.at[b - 1], arow(j0 + b - 1),
                                      ssems[b - 1]).wait()
                pltpu.async_copy(src.at[idx(j0 + NBUF + b - 1)],
                                 bufs.at[b - 1], gsems[b - 1])
        pltpu.make_async_copy(bufs.at[NBUF - 1], arow(j0 + NBUF - 1),
                              ssems[NBUF - 1]).wait()
        pltpu.async_copy(src.at[idx(j0 + 2 * NBUF - 1)], bufs.at[NBUF - 1],
                         gsems[NBUF - 1])
        return 0
    lax.fori_loop(0, nch // NBUF - 1, ring, 0)

    j0 = nch - NBUF
    for b in range(NBUF):
        pltpu.make_async_copy(src.at[idx(j0 + b)], bufs.at[b],
                              gsems[b]).wait()
        pltpu.async_copy(bufs.at[b], arow(j0 + b), ssems[b], add=True)
    for b in range(NBUF):
        pltpu.make_async_copy(bufs.at[b], arow(j0 + b), ssems[b]).wait()


@functools.partial(
    pl.kernel,
    out_type=jax.ShapeDtypeStruct((2, N, H), jnp.float32),
    mesh=plsc.VectorSubcoreMesh(**_MESH),
    compiler_params=pltpu.CompilerParams(use_tc_tiling_on_sc=False),
    scratch_types=[
        pltpu.VMEM((EPT,), jnp.int32),
        pltpu.VMEM((NCH2, CHUNK), jnp.int32),
        pltpu.VMEM((NBUF, CHUNK, H), jnp.float32),
        pltpu.VMEM((RPT, H), jnp.float32),
        pltpu.VMEM((RPT, H), jnp.float32),
        pltpu.VMEM((RPT, H), jnp.float32),
        pltpu.VMEM((16,), jnp.float32),
        pltpu.VMEM((ZR, H), jnp.float32),
        pltpu.VMEM_SHARED((NPAD, H), jnp.float32),
        pltpu.VMEM_SHARED((NPAD, H), jnp.float32),
    ] + [pltpu.SemaphoreType.DMA] * (2 * NBUF),
)
def _agg12(y1_hbm, disb_hbm, b1_hbm, ei_hbm, adj_hbm, out_hbm,
           frm_v, adj_v, bufs, y_v, db_v, z_v, b1_v, zero_v,
           tab_sh, acc_sh, *sems):
    cid = lax.axis_index("c")
    sid = lax.axis_index("s")
    base = sid * RPT
    gsems, ssems = sems[:NBUF], sems[NBUF:]
    z16 = jnp.zeros((16,), jnp.float32)

    pltpu.sync_copy(b1_hbm, b1_v)
    b1vec = b1_v[:]

    def zrow(i, _):
        zero_v[i, :] = z16
        return 0
    lax.fori_loop(0, ZR, zrow, 0)

    pltpu.sync_copy(ei_hbm.at[0, pl.ds(sid * EPT, EPT)], frm_v)
    pltpu.sync_copy(adj_hbm.at[2 * sid], adj_v.at[pl.ds(0, NCH)])
    pltpu.sync_copy(adj_hbm.at[2 * sid + 1], adj_v.at[pl.ds(NCH, NCH)])

    def load_rows(rn):
        pltpu.sync_copy(y1_hbm.at[pl.ds(base, rn)], y_v.at[pl.ds(0, rn)])
        pltpu.sync_copy(disb_hbm.at[pl.ds(base, rn)], db_v.at[pl.ds(0, rn)])
        pltpu.sync_copy(y_v.at[pl.ds(0, rn)], acc_sh.at[pl.ds(base, rn)])

    @pl.when(sid < 15)
    def _():
        load_rows(RPT)

    @pl.when(sid == 15)
    def _():
        load_rows(TAIL)

    plsc.subcore_barrier()

    _ring(y1_hbm, frm_v, adj_v, bufs, gsems, ssems, acc_sh, NCH2, 0)
    plsc.subcore_barrier()

    def phase_z(rn):
        pltpu.sync_copy(acc_sh.at[pl.ds(base, rn)], y_v.at[pl.ds(0, rn)])

        def row(r, _):
            dbr = db_v[r, :]
            h = jnp.maximum(y_v[r, :] * dbr + b1vec, 0.0)
            z_v[r, :] = h * dbr
            return 0
        lax.fori_loop(0, rn, row, 0)

        pltpu.sync_copy(z_v.at[pl.ds(0, rn)], tab_sh.at[pl.ds(base, rn)])

        @pl.when(cid == 0)
        def _():
            pltpu.sync_copy(z_v.at[pl.ds(0, rn)], acc_sh.at[pl.ds(base, rn)])

        @pl.when(cid == 1)
        def _():
            for t in range(rn // ZR):
                pltpu.sync_copy(zero_v, acc_sh.at[pl.ds(base + t * ZR, ZR)])

    @pl.when(sid < 15)
    def _():
        phase_z(RPT)

    @pl.when(sid == 15)
    def _():
        phase_z(TAIL)

    plsc.subcore_barrier()

    _ring(tab_sh, frm_v, adj_v, bufs, gsems, ssems, acc_sh, NCH, cid * NCH)
    plsc.subcore_barrier()

    @pl.when(sid < 15)
    def _():
        pltpu.sync_copy(acc_sh.at[pl.ds(sid * RPT, RPT)],
                        out_hbm.at[cid, pl.ds(sid * RPT, RPT)])

    @pl.when(sid == 15)
    def _():
        pltpu.sync_copy(acc_sh.at[pl.ds(15 * RPT, TAIL)],
                        out_hbm.at[cid, pl.ds(15 * RPT, TAIL)])


BLK = 1000


def _tc1_body(deg_ref, x_ref, w_ref, dis_ref, disb_ref, y_ref):
    deg = deg_ref[:, 0] + deg_ref[:, 1] + 1.0
    dis = lax.rsqrt(deg)
    y = jnp.dot(x_ref[...], w_ref[...], preferred_element_type=jnp.float32)
    dis_ref[...] = dis[:, None]
    disb_ref[...] = jnp.broadcast_to(dis[:, None], disb_ref.shape)
    y_ref[...] = y * dis[:, None]


def _tc3_body(dis_ref, q_ref, b2_ref, w2_ref, out_ref):
    agg = (q_ref[0] + q_ref[1]) * dis_ref[...]
    logits = jnp.dot(agg, w2_ref[...],
                     preferred_element_type=jnp.float32) + b2_ref[...]
    m = jnp.max(logits, axis=1, keepdims=True)
    s = jnp.log(jnp.sum(jnp.exp(logits - m), axis=1, keepdims=True))
    out_ref[...] = logits - m - s


def _tc1(deg2, x, w1t):
    return pl.pallas_call(
        _tc1_body,
        grid=(N // BLK,),
        in_specs=[
            pl.BlockSpec((BLK, 2), lambda i: (i, 0)),
            pl.BlockSpec((BLK, F), lambda i: (i, 0)),
            pl.BlockSpec((F, H), lambda i: (0, 0)),
        ],
        out_specs=[
            pl.BlockSpec((BLK, 1), lambda i: (i, 0)),
            pl.BlockSpec((BLK, H), lambda i: (i, 0)),
            pl.BlockSpec((BLK, H), lambda i: (i, 0)),
        ],
        out_shape=[
            jax.ShapeDtypeStruct((N, 1), jnp.float32),
            jax.ShapeDtypeStruct((N, H), jnp.float32),
            jax.ShapeDtypeStruct((N, H), jnp.float32),
        ],
    )(deg2, x, w1t)


def _tc3(dis, p2, b2, w2t):
    return pl.pallas_call(
        _tc3_body,
        grid=(N // BLK,),
        in_specs=[
            pl.BlockSpec((BLK, 1), lambda i: (i, 0)),
            pl.BlockSpec((2, BLK, H), lambda i: (0, i, 0)),
            pl.BlockSpec((1, L), lambda i: (0, 0)),
            pl.BlockSpec((H, L), lambda i: (0, 0)),
        ],
        out_specs=pl.BlockSpec((BLK, L), lambda i: (i, 0)),
        out_shape=jax.ShapeDtypeStruct((N, L), jnp.float32),
    )(dis, p2, b2, w2t)


def kernel(x, edge_index, W1, b1, W2, b2):
    ei = edge_index.astype(jnp.int32)

    deg_p, adj = _deg_adj(ei)
    dis, disb, y1 = _tc1(deg_p.T, x, W1.T)
    q = _agg12(y1, disb, b1, ei, adj)
    return _tc3(dis, q, b2.reshape(1, L), W2.T)


# speedup vs baseline: 1.1414x; 1.1414x over previous
import functools

import jax
import jax.numpy as jnp
from jax import lax
from jax.experimental import pallas as pl
from jax.experimental.pallas import tpu as pltpu
from jax.experimental.pallas import tpu_sc as plsc

N = 10000
E = 320000
F = 128
H = 16
L = 64

NPAD = 10240
NW = 32
EPW = E // NW
CHUNK = 80
NCH = EPW // CHUNK
RPT = NPAD // 16
NBUF = 5
NRING = NCH // NBUF
ZR = 40
TAIL = N - 15 * RPT

_MESH = dict(core_axis_name="c", subcore_axis_name="s")


@functools.partial(
    pl.kernel,
    out_type=[
        jax.ShapeDtypeStruct((2, N), jnp.float32),
        jax.ShapeDtypeStruct((NW, NCH, CHUNK), jnp.int32),
    ],
    mesh=plsc.VectorSubcoreMesh(**_MESH),
    compiler_params=pltpu.CompilerParams(use_tc_tiling_on_sc=False),
    scratch_types=[
        pltpu.VMEM((EPW,), jnp.int32),
        pltpu.VMEM((EPW,), jnp.int32),
        pltpu.VMEM((NCH, CHUNK), jnp.int32),
        pltpu.VMEM((CHUNK,), jnp.float32),
        pltpu.VMEM((RPT,), jnp.float32),
        pltpu.VMEM_SHARED((NPAD,), jnp.float32),
        pltpu.SemaphoreType.DMA,
    ],
)
def _deg_adj(ei_hbm, deg_hbm, adj_hbm,
             frm_v, to_v, adj_v, ones_v, zero_v, acc_sh, dsem):
    cid = lax.axis_index("c")
    sid = lax.axis_index("s")
    wid = cid * 16 + sid
    z16 = jnp.zeros((16,), jnp.float32)
    o16 = jnp.ones((16,), jnp.float32)

    def fill_z(i, _):
        zero_v[pl.ds(i * 16, 16)] = z16
        return 0
    lax.fori_loop(0, RPT // 16, fill_z, 0)

    def fill_o(i, _):
        ones_v[pl.ds(i * 16, 16)] = o16
        return 0
    lax.fori_loop(0, CHUNK // 16, fill_o, 0)

    pltpu.sync_copy(zero_v, acc_sh.at[pl.ds(sid * RPT, RPT)])
    pltpu.sync_copy(ei_hbm.at[0, pl.ds(wid * EPW, EPW)], frm_v)
    pltpu.sync_copy(ei_hbm.at[1, pl.ds(wid * EPW, EPW)], to_v)
    plsc.subcore_barrier()

    def chunk(j, _):
        dump = N + ((wid * NCH + j) * 7) % (NPAD - N)

        def vec(k, _):
            f16 = frm_v[pl.ds(j * CHUNK + k * 16, 16)]
            t16 = to_v[pl.ds(j * CHUNK + k * 16, 16)]
            adj_v[j, pl.ds(k * 16, 16)] = jnp.where(f16 != t16, t16, dump)
            return 0
        lax.fori_loop(0, CHUNK // 16, vec, 0)
        pltpu.async_copy(ones_v, acc_sh.at[adj_v.at[j]], dsem, add=True)
        return 0
    lax.fori_loop(0, NCH, chunk, 0)

    pltpu.sync_copy(adj_v, adj_hbm.at[wid])

    def drain(j, _):
        pltpu.make_async_copy(ones_v, acc_sh.at[adj_v.at[j]], dsem).wait()
        return 0
    lax.fori_loop(0, NCH, drain, 0)
    plsc.subcore_barrier()

    @pl.when(sid < 15)
    def _():
        pltpu.sync_copy(acc_sh.at[pl.ds(sid * RPT, RPT)],
                        deg_hbm.at[cid, pl.ds(sid * RPT, RPT)])

    @pl.when(sid == 15)
    def _():
        pltpu.sync_copy(acc_sh.at[pl.ds(15 * RPT, TAIL)],
                        deg_hbm.at[cid, pl.ds(15 * RPT, TAIL)])


def _ring(src, idx, arow, bufs, gsems, ssems, nch):
    for b in range(NBUF):
        pltpu.async_copy(src.at[idx(b)], bufs.at[b], gsems[b])

    def ring(g, _):
        j0 = g * NBUF
        for b in range(NBUF):
            pltpu.make_async_copy(src.at[idx(j0 + b)], bufs.at[b],
                                  gsems[b]).wait()
            pltpu.async_copy(bufs.at[b], arow(j0 + b), ssems[b], add=True)
            if b > 0:
                pltpu.make_async_copy(bufs.at[b - 1], arow(j0 + b - 1),
                                      ssems[b - 1]).wait()
                pltpu.async_copy(src.at[idx(j0 + NBUF + b - 1)],
                                 bufs.at[b - 1], gsems[b - 1])
        pltpu.make_async_copy(bufs.at[NBUF - 1], arow(j0 + NBUF - 1),
                              ssems[NBUF - 1]).wait()
        pltpu.async_copy(src.at[idx(j0 + 2 * NBUF - 1)], bufs.at[NBUF - 1],
                         gsems[NBUF - 1])
        return 0
    lax.fori_loop(0, nch // NBUF - 1, ring, 0)

    j0 = nch - NBUF
    for b in range(NBUF):
        pltpu.make_async_copy(src.at[idx(j0 + b)], bufs.at[b],
                              gsems[b]).wait()
        pltpu.async_copy(bufs.at[b], arow(j0 + b), ssems[b], add=True)
    for b in range(NBUF):
        pltpu.make_async_copy(bufs.at[b], arow(j0 + b), ssems[b]).wait()


@functools.partial(
    pl.kernel,
    out_type=jax.ShapeDtypeStruct((2, N, H), jnp.float32),
    mesh=plsc.VectorSubcoreMesh(**_MESH),
    compiler_params=pltpu.CompilerParams(use_tc_tiling_on_sc=False),
    scratch_types=[
        pltpu.VMEM((EPW,), jnp.int32),
        pltpu.VMEM((NCH, CHUNK), jnp.int32),
        pltpu.VMEM((NBUF, CHUNK, H), jnp.float32),
        pltpu.VMEM((ZR, H), jnp.float32),
        pltpu.VMEM_SHARED((NPAD, H), jnp.float32),
    ] + [pltpu.SemaphoreType.DMA] * (2 * NBUF),
)
def _agg(y_hbm, ei_hbm, adj_hbm, out_hbm,
         frm_v, adj_v, bufs, zero_v, acc_sh, *sems):
    cid = lax.axis_index("c")
    sid = lax.axis_index("s")
    wid = cid * 16 + sid
    gsems, ssems = sems[:NBUF], sems[NBUF:]
    z16 = jnp.zeros((16,), jnp.float32)

    def zrow(i, _):
        zero_v[i, :] = z16
        return 0
    lax.fori_loop(0, ZR, zrow, 0)

    for t in range(RPT // ZR):
        pltpu.sync_copy(zero_v, acc_sh.at[pl.ds(sid * RPT + t * ZR, ZR)])
    pltpu.sync_copy(ei_hbm.at[0, pl.ds(wid * EPW, EPW)], frm_v)
    pltpu.sync_copy(adj_hbm.at[wid], adj_v)
    plsc.subcore_barrier()

    _ring(y_hbm,
          lambda j: frm_v.at[pl.ds(j * CHUNK, CHUNK)],
          lambda j: acc_sh.at[adj_v.at[j]],
          bufs, gsems, ssems, NCH)

    plsc.subcore_barrier()

    @pl.when(sid < 15)
    def _():
        pltpu.sync_copy(acc_sh.at[pl.ds(sid * RPT, RPT)],
                        out_hbm.at[cid, pl.ds(sid * RPT, RPT)])

    @pl.when(sid == 15)
    def _():
        pltpu.sync_copy(acc_sh.at[pl.ds(15 * RPT, TAIL)],
                        out_hbm.at[cid, pl.ds(15 * RPT, TAIL)])


@functools.partial(
    pl.kernel,
    out_type=jax.ShapeDtypeStruct((2, N, H), jnp.float32),
    mesh=plsc.VectorSubcoreMesh(**_MESH),
    compiler_params=pltpu.CompilerParams(use_tc_tiling_on_sc=False),
    scratch_types=[
        pltpu.VMEM((EPW,), jnp.int32),
        pltpu.VMEM((NCH, CHUNK), jnp.int32),
        pltpu.VMEM((NBUF, CHUNK, H), jnp.float32),
        pltpu.VMEM((RPT, H), jnp.float32),
        pltpu.VMEM((RPT, H), jnp.float32),
        pltpu.VMEM((RPT, H), jnp.float32),
        pltpu.VMEM((RPT, H), jnp.float32),
        pltpu.VMEM((RPT, H), jnp.float32),
        pltpu.VMEM((16,), jnp.float32),
        pltpu.VMEM((ZR, H), jnp.float32),
        pltpu.VMEM_SHARED((NPAD, H), jnp.float32),
        pltpu.VMEM_SHARED((NPAD, H), jnp.float32),
    ] + [pltpu.SemaphoreType.DMA] * (2 * NBUF),
)
def _agg2(p1_hbm, y1_hbm, disb_hbm, b1_hbm, ei_hbm, adj_hbm, out_hbm,
          frm_v, adj_v, bufs, pa_v, pb_v, y_v, z_v, db_v, b1_v, zero_v,
          zsp_sh, acc_sh, *sems):
    cid = lax.axis_index("c")
    sid = lax.axis_index("s")
    wid = cid * 16 + sid
    base = sid * RPT
    gsems, ssems = sems[:NBUF], sems[NBUF:]
    z16 = jnp.zeros((16,), jnp.float32)

    pltpu.sync_copy(b1_hbm, b1_v)
    b1vec = b1_v[:]

    def zrow(i, _):
        zero_v[i, :] = z16
        return 0
    lax.fori_loop(0, ZR, zrow, 0)

    def phase1(rn):
        pltpu.sync_copy(p1_hbm.at[0, pl.ds(base, rn)], pa_v.at[pl.ds(0, rn)])
        pltpu.sync_copy(p1_hbm.at[1, pl.ds(base, rn)], pb_v.at[pl.ds(0, rn)])
        pltpu.sync_copy(y1_hbm.at[pl.ds(base, rn)], y_v.at[pl.ds(0, rn)])
        pltpu.sync_copy(disb_hbm.at[pl.ds(base, rn)], db_v.at[pl.ds(0, rn)])

        def row(r, _):
            dbr = db_v[r, :]
            p16 = pa_v[r, :] + pb_v[r, :] + y_v[r, :]
            h = jnp.maximum(p16 * dbr + b1vec, 0.0)
            z_v[r, :] = h * dbr
            return 0
        lax.fori_loop(0, rn, row, 0)

        pltpu.sync_copy(z_v.at[pl.ds(0, rn)], zsp_sh.at[pl.ds(base, rn)])

        @pl.when(cid == 0)
        def _():
            pltpu.sync_copy(z_v.at[pl.ds(0, rn)], acc_sh.at[pl.ds(base, rn)])

        @pl.when(cid == 1)
        def _():
            for t in range(rn // ZR):
                pltpu.sync_copy(zero_v, acc_sh.at[pl.ds(base + t * ZR, ZR)])

    @pl.when(sid < 15)
    def _():
        phase1(RPT)

    @pl.when(sid == 15)
    def _():
        phase1(TAIL)

    pltpu.sync_copy(ei_hbm.at[0, pl.ds(wid * EPW, EPW)], frm_v)
    pltpu.sync_copy(adj_hbm.at[wid], adj_v)
    plsc.subcore_barrier()

    _ring(zsp_sh,
          lambda j: frm_v.at[pl.ds(j * CHUNK, CHUNK)],
          lambda j: acc_sh.at[adj_v.at[j]],
          bufs, gsems, ssems, NCH)

    plsc.subcore_barrier()

    @pl.when(sid < 15)
    def _():
        pltpu.sync_copy(acc_sh.at[pl.ds(sid * RPT, RPT)],
                        out_hbm.at[cid, pl.ds(sid * RPT, RPT)])

    @pl.when(sid == 15)
    def _():
        pltpu.sync_copy(acc_sh.at[pl.ds(15 * RPT, TAIL)],
                        out_hbm.at[cid, pl.ds(15 * RPT, TAIL)])


BLK = 1000


def _tc1_body(deg_ref, x_ref, w_ref, dis_ref, disb_ref, y_ref):
    deg = deg_ref[:, 0] + deg_ref[:, 1] + 1.0
    dis = lax.rsqrt(deg)
    y = jnp.dot(x_ref[...], w_ref[...], preferred_element_type=jnp.float32)
    dis_ref[...] = dis[:, None]
    disb_ref[...] = jnp.broadcast_to(dis[:, None], disb_ref.shape)
    y_ref[...] = y * dis[:, None]


def _tc3_body(dis_ref, q_ref, b2_ref, w2_ref, out_ref):
    agg = (q_ref[0] + q_ref[1]) * dis_ref[...]
    logits = jnp.dot(agg, w2_ref[...],
                     preferred_element_type=jnp.float32) + b2_ref[...]
    m = jnp.max(logits, axis=1, keepdims=True)
    s = jnp.log(jnp.sum(jnp.exp(logits - m), axis=1, keepdims=True))
    out_ref[...] = logits - m - s


def _tc1(deg2, x, w1t):
    return pl.pallas_call(
        _tc1_body,
        grid=(N // BLK,),
        in_specs=[
            pl.BlockSpec((BLK, 2), lambda i: (i, 0)),
            pl.BlockSpec((BLK, F), lambda i: (i, 0)),
            pl.BlockSpec((F, H), lambda i: (0, 0)),
        ],
        out_specs=[
            pl.BlockSpec((BLK, 1), lambda i: (i, 0)),
            pl.BlockSpec((BLK, H), lambda i: (i, 0)),
            pl.BlockSpec((BLK, H), lambda i: (i, 0)),
        ],
        out_shape=[
            jax.ShapeDtypeStruct((N, 1), jnp.float32),
            jax.ShapeDtypeStruct((N, H), jnp.float32),
            jax.ShapeDtypeStruct((N, H), jnp.float32),
        ],
    )(deg2, x, w1t)


def _tc3(dis, p2, b2, w2t):
    return pl.pallas_call(
        _tc3_body,
        grid=(N // BLK,),
        in_specs=[
            pl.BlockSpec((BLK, 1), lambda i: (i, 0)),
            pl.BlockSpec((2, BLK, H), lambda i: (0, i, 0)),
            pl.BlockSpec((1, L), lambda i: (0, 0)),
            pl.BlockSpec((H, L), lambda i: (0, 0)),
        ],
        out_specs=pl.BlockSpec((BLK, L), lambda i: (i, 0)),
        out_shape=jax.ShapeDtypeStruct((N, L), jnp.float32),
    )(dis, p2, b2, w2t)


def kernel(x, edge_index, W1, b1, W2, b2):
    ei = edge_index.astype(jnp.int32)

    deg_p, adj = _deg_adj(ei)
    dis, disb, y1 = _tc1(deg_p.T, x, W1.T)
    p1 = _agg(y1, ei, adj)
    q = _agg2(p1, y1, disb, b1, ei, adj)
    return _tc3(dis, q, b2.reshape(1, L), W2.T)
